# baseline (speedup 1.0000x reference)
import functools

import jax
import jax.numpy as jnp
from jax import lax
from jax.experimental import pallas as pl
from jax.experimental.pallas import tpu as pltpu
from jax.experimental.pallas import tpu_sc as plsc

_NC = 2
_NS = 16
_NW = _NC * _NS
_EK = 80
_BLK = 2000
_B = 64


def _sc_mesh():
    return plsc.VectorSubcoreMesh(core_axis_name="c", subcore_axis_name="s",
                                  num_cores=_NC, num_subcores=_NS)


def _zero_1d(ref, n):
    def body(i, c):
        ref[pl.ds(i * 16, 16)] = jnp.zeros((16,), ref.dtype)
        return c
    lax.fori_loop(0, n // 16, body, 0)


def _zero_2d(ref, rows, cols):
    nk = cols // 16
    def body(t, c):
        r = t // nk
        k = t % nk
        ref[r, pl.ds(k * 16, 16)] = jnp.zeros((16,), ref.dtype)
        return c
    lax.fori_loop(0, rows * nk, body, 0)


def _make_deg(E, N):
    EPW = E // _NW
    CH = EPW // _EK
    ZR = 2000
    NZ = N // ZR

    @functools.partial(
        pl.kernel,
        out_type=jax.ShapeDtypeStruct((_NC * N,), jnp.float32),
        mesh=_sc_mesh(),
        scratch_types=[
            pltpu.VMEM((CH, _EK), jnp.int32),
            pltpu.VMEM((_EK,), jnp.float32),
            pltpu.VMEM((ZR,), jnp.float32),
            pltpu.VMEM_SHARED((N,), jnp.float32),
            pltpu.SemaphoreType.DMA,
            pltpu.SemaphoreType.DMA,
        ],
    )
    def deg_kernel(dst_hbm, deg_hbm, idx_v, ones_v, buf_v, acc_sh, sem0, sem1):
        cid = lax.axis_index("c")
        sid = lax.axis_index("s")
        wid = sid * _NC + cid
        pltpu.sync_copy(dst_hbm.at[wid], idx_v)
        for i in range(_EK // 16):
            ones_v[pl.ds(i * 16, 16)] = jnp.ones((16,), jnp.float32)

        @pl.when(sid < NZ)
        def _():
            _zero_1d(buf_v, ZR)
            pltpu.sync_copy(buf_v, acc_sh.at[pl.ds(sid * ZR, ZR)])

        plsc.subcore_barrier()
        sems = (sem0, sem1)

        def fire(g, p):
            pltpu.async_copy(ones_v, acc_sh.at[idx_v.at[g]], sems[p], add=True)

        def drain(p):
            pltpu.make_async_copy(deg_hbm.at[pl.ds(0, _EK)], ones_v,
                                  sems[p]).wait()

        fire(0, 0)

        def pair(k, c):
            g = 2 * k
            fire(g + 1, 1)
            drain(0)
            fire(g + 2, 0)
            drain(1)
            return c

        lax.fori_loop(0, (CH - 1) // 2, pair, 0)
        drain(0)
        plsc.subcore_barrier()

        @pl.when(sid < NZ)
        def _():
            pltpu.sync_copy(acc_sh.at[pl.ds(sid * ZR, ZR)], buf_v)
            pltpu.sync_copy(buf_v, deg_hbm.at[pl.ds(cid * N + sid * ZR, ZR)])

    return deg_kernel


def _make_scatter(E, N, D):
    EPW = E // _NW
    CH = EPW // _EK
    assert CH % 3 == 2 and CH >= 8
    ZT = 1000
    NZ = N // ZT

    @functools.partial(
        pl.kernel,
        out_type=jax.ShapeDtypeStruct((_NC * N, D), jnp.float32),
        mesh=_sc_mesh(),
        scratch_types=[
            pltpu.VMEM((EPW,), jnp.int32),
            pltpu.VMEM((3, _EK), jnp.int32),
            pltpu.VMEM((3, _EK, D), jnp.float32),
            pltpu.VMEM_SHARED((N, D), jnp.float32),
        ] + [pltpu.SemaphoreType.DMA] * 6,
    )
    def scat_kernel(src_hbm, dst_hbm, zero_hbm, y_hbm, out_hbm,
                    sidx_v, didx_v, rows_v, acc_sh,
                    g0s, g1s, g2s, s0s, s1s, s2s):
        cid = lax.axis_index("c")
        sid = lax.axis_index("s")
        wid = sid * _NC + cid
        pltpu.sync_copy(src_hbm.at[pl.ds(wid * EPW, EPW)], sidx_v)

        @pl.when(jnp.logical_and(sid < NZ, cid == 0))
        def _():
            pltpu.sync_copy(y_hbm.at[pl.ds(sid * ZT, ZT)],
                            acc_sh.at[pl.ds(sid * ZT, ZT)])

        @pl.when(jnp.logical_and(sid < NZ, cid != 0))
        def _():
            pltpu.sync_copy(zero_hbm.at[pl.ds(sid * ZT, ZT)],
                            acc_sh.at[pl.ds(sid * ZT, ZT)])

        plsc.subcore_barrier()
        gsems = (g0s, g1s, g2s)
        ssems = (s0s, s1s, s2s)

        def fire(g, s):
            off = wid * EPW + g * _EK
            pltpu.async_copy(dst_hbm.at[pl.ds(off, _EK)], didx_v.at[s],
                             gsems[s])
            pltpu.async_copy(y_hbm.at[sidx_v.at[pl.ds(g * _EK, _EK)]],
                             rows_v.at[s], gsems[s])

        def wait_fire(s):
            pltpu.make_async_copy(dst_hbm.at[pl.ds(0, _EK)], didx_v.at[s],
                                  gsems[s]).wait()
            pltpu.make_async_copy(y_hbm.at[pl.ds(0, _EK)], rows_v.at[s],
                                  gsems[s]).wait()

        def scat(s):
            pltpu.async_copy(rows_v.at[s], acc_sh.at[didx_v.at[s]], ssems[s],
                             add=True)

        def wait_scat(s):
            pltpu.make_async_copy(y_hbm.at[pl.ds(0, _EK)], rows_v.at[s],
                                  ssems[s]).wait()

        for s in range(3):
            fire(s, s)

        NB = CH // 3 - 1

        def body(k, c):
            g0 = 3 * k
            for s in range(3):
                wait_fire(s)
                scat(s)
            for s in range(3):
                wait_scat(s)
                fire(g0 + 3 + s, s)
            return c

        lax.fori_loop(0, NB, body, 0)

        t0 = 3 * NB
        for s in range(3):
            wait_fire(s)
            scat(s)
        for g, s in ((t0 + 3, 0), (t0 + 4, 1)):
            wait_scat(s)
            fire(g, s)
        for s in range(2):
            wait_fire(s)
            scat(s)
        for s in range(3):
            wait_scat(s)
        plsc.subcore_barrier()

        @pl.when(sid < NZ)
        def _():
            r0 = sid * ZT
            pltpu.sync_copy(acc_sh.at[pl.ds(r0, ZT)],
                            out_hbm.at[pl.ds(cid * N + r0, ZT)])

    return scat_kernel


def _mm1(x, W1, degT):
    N, D = x.shape
    H = W1.shape[1]

    def body(x_ref, w_ref, dg_ref, y_ref, dis_ref):
        dis = lax.rsqrt(dg_ref[:, 0:1] + dg_ref[:, 1:2] + 1.0)
        y_ref[...] = dis * jnp.dot(x_ref[...], w_ref[...],
                                   preferred_element_type=jnp.float32)
        dis_ref[...] = dis

    return pl.pallas_call(
        body,
        grid=(N // _BLK,),
        in_specs=[pl.BlockSpec((_BLK, D), lambda i: (i, 0)),
                  pl.BlockSpec((D, H), lambda i: (0, 0)),
                  pl.BlockSpec((_BLK, 2), lambda i: (i, 0))],
        out_specs=[pl.BlockSpec((_BLK, H), lambda i: (i, 0)),
                   pl.BlockSpec((_BLK, 1), lambda i: (i, 0))],
        out_shape=[jax.ShapeDtypeStruct((N, H), jnp.float32),
                   jax.ShapeDtypeStruct((N, 1), jnp.float32)],
    )(x, W1, degT)


def _mm2(s1, dis, b1, W2):
    N2, H = s1.shape
    N = N2 // 2
    nb = N // _BLK

    def body(pa_ref, pb_ref, dis_ref, b_ref, w_ref, o_ref):
        d = dis_ref[...]
        h = d * (pa_ref[...] + pb_ref[...]) + b_ref[...]
        h = jnp.maximum(h, 0.0)
        o_ref[...] = d * jnp.dot(h, w_ref[...],
                                 preferred_element_type=jnp.float32)

    return pl.pallas_call(
        body,
        grid=(nb,),
        in_specs=[pl.BlockSpec((_BLK, H), lambda i: (i, 0)),
                  pl.BlockSpec((_BLK, H), lambda i: (i + nb, 0)),
                  pl.BlockSpec((_BLK, 1), lambda i: (i, 0)),
                  pl.BlockSpec((1, H), lambda i: (0, 0)),
                  pl.BlockSpec((H, H), lambda i: (0, 0))],
        out_specs=pl.BlockSpec((_BLK, H), lambda i: (i, 0)),
        out_shape=jax.ShapeDtypeStruct((N, H), jnp.float32),
    )(s1, s1, dis, b1, W2)


def _mm3(s2, dis, b2, Wpad, batch2, blin2, C):
    N2, H = s2.shape
    N = N2 // 2
    nb = N // _BLK

    def body(qa_ref, qb_ref, dis_ref, b_ref, w_ref, bat_ref, bl_ref,
             o_ref, acc_ref):
        i = pl.program_id(0)
        d = dis_ref[...]
        h = d * (qa_ref[...] + qb_ref[...]) + b_ref[...]
        z = jnp.dot(h, w_ref[...], preferred_element_type=jnp.float32)
        z = z + (lax.broadcasted_iota(jnp.int32, (1, H), 1) == C
                 ).astype(jnp.float32)
        bm = (bat_ref[...] == lax.broadcasted_iota(jnp.int32, (_BLK, _B), 1)
              ).astype(jnp.float32)
        part = lax.dot_general(bm, z, (((0,), (0,)), ((), ())),
                               preferred_element_type=jnp.float32)

        @pl.when(i == 0)
        def _():
            acc_ref[...] = part

        @pl.when(i > 0)
        def _():
            acc_ref[...] = acc_ref[...] + part

        @pl.when(i == nb - 1)
        def _():
            cnt = jnp.maximum(acc_ref[:, C:C + 1], 1.0)
            o_ref[...] = acc_ref[:, :C] / cnt + bl_ref[...]

    return pl.pallas_call(
        body,
        grid=(nb,),
        in_specs=[pl.BlockSpec((_BLK, H), lambda i: (i, 0)),
                  pl.BlockSpec((_BLK, H), lambda i: (i + nb, 0)),
                  pl.BlockSpec((_BLK, 1), lambda i: (i, 0)),
                  pl.BlockSpec((1, H), lambda i: (0, 0)),
                  pl.BlockSpec((H, H), lambda i: (0, 0)),
                  pl.BlockSpec((_BLK, 1), lambda i: (i, 0)),
                  pl.BlockSpec((1, C), lambda i: (0, 0))],
        out_specs=pl.BlockSpec((_B, C), lambda i: (0, 0)),
        out_shape=jax.ShapeDtypeStruct((_B, C), jnp.float32),
        scratch_shapes=[pltpu.VMEM((_B, H), jnp.float32)],
    )(s2, s2, dis, b2, Wpad, batch2, blin2)


def kernel(x, edge_index, batch, W1, b1, W2, b2, Wlin, blin):
    N, D = x.shape
    E = edge_index.shape[1]
    H = W1.shape[1]
    C = Wlin.shape[1]
    assert N % _BLK == 0 and E % (_NW * _EK) == 0 and D == H and C < H

    CH = E // (_NW * _EK)
    src = edge_index[0]
    dst = edge_index[1]
    dst3 = dst.reshape(_NW, CH, _EK)
    zeros = jnp.zeros((N, D), jnp.float32)

    deg_fn = _make_deg(E, N)
    scat_fn = _make_scatter(E, N, D)

    deg2 = deg_fn(dst3).reshape(_NC, N).T
    y1, dis = _mm1(x, W1, deg2)
    s1 = scat_fn(src, dst, zeros, y1)
    y2 = _mm2(s1, dis, b1.reshape(1, H), W2)
    s2 = scat_fn(src, dst, zeros, y2)
    out = _mm3(s2, dis, b2.reshape(1, H), jnp.pad(Wlin, ((0, 0), (0, H - C))),
               batch.reshape(N, 1), blin.reshape(1, C), C)
    return out


# speedup vs baseline: 1.0124x; 1.0124x over previous
import functools

import jax
import jax.numpy as jnp
from jax import lax
from jax.experimental import pallas as pl
from jax.experimental.pallas import tpu as pltpu
from jax.experimental.pallas import tpu_sc as plsc

_NC = 2
_NS = 16
_NW = _NC * _NS
_EK = 80
_BLK = 2000
_B = 64


def _sc_mesh():
    return plsc.VectorSubcoreMesh(core_axis_name="c", subcore_axis_name="s",
                                  num_cores=_NC, num_subcores=_NS)


def _zero_1d(ref, n):
    def body(i, c):
        ref[pl.ds(i * 16, 16)] = jnp.zeros((16,), ref.dtype)
        return c
    lax.fori_loop(0, n // 16, body, 0)


def _zero_2d(ref, rows, cols):
    nk = cols // 16
    def body(t, c):
        r = t // nk
        k = t % nk
        ref[r, pl.ds(k * 16, 16)] = jnp.zeros((16,), ref.dtype)
        return c
    lax.fori_loop(0, rows * nk, body, 0)


def _make_deg(E, N):
    EPW = E // _NW
    CH = EPW // _EK
    ZR = 2000
    NZ = N // ZR

    @functools.partial(
        pl.kernel,
        out_type=jax.ShapeDtypeStruct((_NC * N,), jnp.float32),
        mesh=_sc_mesh(),
        scratch_types=[
            pltpu.VMEM((CH, _EK), jnp.int32),
            pltpu.VMEM((_EK,), jnp.float32),
            pltpu.VMEM((ZR,), jnp.float32),
            pltpu.VMEM_SHARED((N,), jnp.float32),
            pltpu.SemaphoreType.DMA,
            pltpu.SemaphoreType.DMA,
        ],
    )
    def deg_kernel(dst_hbm, deg_hbm, idx_v, ones_v, buf_v, acc_sh, sem0, sem1):
        cid = lax.axis_index("c")
        sid = lax.axis_index("s")
        wid = sid * _NC + cid
        pltpu.sync_copy(dst_hbm.at[wid], idx_v)
        for i in range(_EK // 16):
            ones_v[pl.ds(i * 16, 16)] = jnp.ones((16,), jnp.float32)

        @pl.when(sid < NZ)
        def _():
            _zero_1d(buf_v, ZR)
            pltpu.sync_copy(buf_v, acc_sh.at[pl.ds(sid * ZR, ZR)])

        plsc.subcore_barrier()
        sems = (sem0, sem1)

        def fire(g, p):
            pltpu.async_copy(ones_v, acc_sh.at[idx_v.at[g]], sems[p], add=True)

        def drain(p):
            pltpu.make_async_copy(deg_hbm.at[pl.ds(0, _EK)], ones_v,
                                  sems[p]).wait()

        fire(0, 0)

        def pair(k, c):
            g = 2 * k
            fire(g + 1, 1)
            drain(0)
            fire(g + 2, 0)
            drain(1)
            return c

        lax.fori_loop(0, (CH - 1) // 2, pair, 0)
        drain(0)
        plsc.subcore_barrier()

        @pl.when(sid < NZ)
        def _():
            pltpu.sync_copy(acc_sh.at[pl.ds(sid * ZR, ZR)], buf_v)
            pltpu.sync_copy(buf_v, deg_hbm.at[pl.ds(cid * N + sid * ZR, ZR)])

    return deg_kernel


def _make_scatter(E, N, D):
    EPW = E // _NW
    CH = EPW // _EK
    assert CH % 2 == 1 and CH >= 3
    ZT = 1000
    NZ = N // ZT

    @functools.partial(
        pl.kernel,
        out_type=jax.ShapeDtypeStruct((_NC * N, D), jnp.float32),
        mesh=_sc_mesh(),
        scratch_types=[
            pltpu.VMEM((EPW,), jnp.int32),
            pltpu.VMEM((CH, _EK), jnp.int32),
            pltpu.VMEM((2, _EK, D), jnp.float32),
            pltpu.VMEM_SHARED((N, D), jnp.float32),
            pltpu.SemaphoreType.DMA,
            pltpu.SemaphoreType.DMA,
        ],
    )
    def scat_kernel(src_hbm, dst3_hbm, zero_hbm, y_hbm, out_hbm,
                    sidx_v, didx_v, rows_v, acc_sh, sem0, sem1):
        cid = lax.axis_index("c")
        sid = lax.axis_index("s")
        wid = sid * _NC + cid
        pltpu.sync_copy(src_hbm.at[pl.ds(wid * EPW, EPW)], sidx_v)
        pltpu.sync_copy(dst3_hbm.at[wid], didx_v)

        @pl.when(jnp.logical_and(sid < NZ, cid == 0))
        def _():
            pltpu.sync_copy(y_hbm.at[pl.ds(sid * ZT, ZT)],
                            acc_sh.at[pl.ds(sid * ZT, ZT)])

        @pl.when(jnp.logical_and(sid < NZ, cid != 0))
        def _():
            pltpu.sync_copy(zero_hbm.at[pl.ds(sid * ZT, ZT)],
                            acc_sh.at[pl.ds(sid * ZT, ZT)])

        plsc.subcore_barrier()
        sems = (sem0, sem1)

        def fire(g, p):
            pltpu.async_copy(y_hbm.at[sidx_v.at[pl.ds(g * _EK, _EK)]],
                             rows_v.at[p], sems[p])

        def consume(g, p):
            pltpu.make_async_copy(y_hbm.at[pl.ds(0, _EK)], rows_v.at[p],
                                  sems[p]).wait()
            pltpu.sync_copy(rows_v.at[p], acc_sh.at[didx_v.at[g]], add=True)

        fire(0, 0)

        def pair(k, c):
            g = 2 * k
            fire(g + 1, 1)
            consume(g, 0)
            fire(g + 2, 0)
            consume(g + 1, 1)
            return c

        lax.fori_loop(0, (CH - 1) // 2, pair, 0)
        consume(CH - 1, 0)
        plsc.subcore_barrier()

        @pl.when(sid < NZ)
        def _():
            r0 = sid * ZT
            pltpu.sync_copy(acc_sh.at[pl.ds(r0, ZT)],
                            out_hbm.at[pl.ds(cid * N + r0, ZT)])

    return scat_kernel


def _mm1(x, W1, degT):
    N, D = x.shape
    H = W1.shape[1]

    def body(x_ref, w_ref, dg_ref, y_ref, dis_ref):
        dis = lax.rsqrt(dg_ref[:, 0:1] + dg_ref[:, 1:2] + 1.0)
        y_ref[...] = dis * jnp.dot(x_ref[...], w_ref[...],
                                   preferred_element_type=jnp.float32)
        dis_ref[...] = dis

    return pl.pallas_call(
        body,
        grid=(N // _BLK,),
        in_specs=[pl.BlockSpec((_BLK, D), lambda i: (i, 0)),
                  pl.BlockSpec((D, H), lambda i: (0, 0)),
                  pl.BlockSpec((_BLK, 2), lambda i: (i, 0))],
        out_specs=[pl.BlockSpec((_BLK, H), lambda i: (i, 0)),
                   pl.BlockSpec((_BLK, 1), lambda i: (i, 0))],
        out_shape=[jax.ShapeDtypeStruct((N, H), jnp.float32),
                   jax.ShapeDtypeStruct((N, 1), jnp.float32)],
    )(x, W1, degT)


def _mm2(s1, dis, b1, W2):
    N2, H = s1.shape
    N = N2 // 2
    nb = N // _BLK

    def body(pa_ref, pb_ref, dis_ref, b_ref, w_ref, o_ref):
        d = dis_ref[...]
        h = d * (pa_ref[...] + pb_ref[...]) + b_ref[...]
        h = jnp.maximum(h, 0.0)
        o_ref[...] = d * jnp.dot(h, w_ref[...],
                                 preferred_element_type=jnp.float32)

    return pl.pallas_call(
        body,
        grid=(nb,),
        in_specs=[pl.BlockSpec((_BLK, H), lambda i: (i, 0)),
                  pl.BlockSpec((_BLK, H), lambda i: (i + nb, 0)),
                  pl.BlockSpec((_BLK, 1), lambda i: (i, 0)),
                  pl.BlockSpec((1, H), lambda i: (0, 0)),
                  pl.BlockSpec((H, H), lambda i: (0, 0))],
        out_specs=pl.BlockSpec((_BLK, H), lambda i: (i, 0)),
        out_shape=jax.ShapeDtypeStruct((N, H), jnp.float32),
    )(s1, s1, dis, b1, W2)


def _mm3(s2, dis, b2, Wpad, batch2, blin2, C):
    N2, H = s2.shape
    N = N2 // 2
    nb = N // _BLK

    def body(qa_ref, qb_ref, dis_ref, b_ref, w_ref, bat_ref, bl_ref,
             o_ref, acc_ref):
        i = pl.program_id(0)
        d = dis_ref[...]
        h = d * (qa_ref[...] + qb_ref[...]) + b_ref[...]
        z = jnp.dot(h, w_ref[...], preferred_element_type=jnp.float32)
        z = z + (lax.broadcasted_iota(jnp.int32, (1, H), 1) == C
                 ).astype(jnp.float32)
        bm = (bat_ref[...] == lax.broadcasted_iota(jnp.int32, (_BLK, _B), 1)
              ).astype(jnp.float32)
        part = lax.dot_general(bm, z, (((0,), (0,)), ((), ())),
                               preferred_element_type=jnp.float32)

        @pl.when(i == 0)
        def _():
            acc_ref[...] = part

        @pl.when(i > 0)
        def _():
            acc_ref[...] = acc_ref[...] + part

        @pl.when(i == nb - 1)
        def _():
            cnt = jnp.maximum(acc_ref[:, C:C + 1], 1.0)
            o_ref[...] = acc_ref[:, :C] / cnt + bl_ref[...]

    return pl.pallas_call(
        body,
        grid=(nb,),
        in_specs=[pl.BlockSpec((_BLK, H), lambda i: (i, 0)),
                  pl.BlockSpec((_BLK, H), lambda i: (i + nb, 0)),
                  pl.BlockSpec((_BLK, 1), lambda i: (i, 0)),
                  pl.BlockSpec((1, H), lambda i: (0, 0)),
                  pl.BlockSpec((H, H), lambda i: (0, 0)),
                  pl.BlockSpec((_BLK, 1), lambda i: (i, 0)),
                  pl.BlockSpec((1, C), lambda i: (0, 0))],
        out_specs=pl.BlockSpec((_B, C), lambda i: (0, 0)),
        out_shape=jax.ShapeDtypeStruct((_B, C), jnp.float32),
        scratch_shapes=[pltpu.VMEM((_B, H), jnp.float32)],
    )(s2, s2, dis, b2, Wpad, batch2, blin2)


def kernel(x, edge_index, batch, W1, b1, W2, b2, Wlin, blin):
    N, D = x.shape
    E = edge_index.shape[1]
    H = W1.shape[1]
    C = Wlin.shape[1]
    assert N % _BLK == 0 and E % (_NW * _EK) == 0 and D == H and C < H

    CH = E // (_NW * _EK)
    src = edge_index[0]
    dst = edge_index[1]
    dst3 = dst.reshape(_NW, CH, _EK)
    zeros = jnp.zeros((N, D), jnp.float32)

    deg_fn = _make_deg(E, N)
    scat_fn = _make_scatter(E, N, D)

    deg2 = deg_fn(dst3).reshape(_NC, N).T
    y1, dis = _mm1(x, W1, deg2)
    s1 = scat_fn(src, dst3, zeros, y1)
    y2 = _mm2(s1, dis, b1.reshape(1, H), W2)
    s2 = scat_fn(src, dst3, zeros, y2)
    out = _mm3(s2, dis, b2.reshape(1, H), jnp.pad(Wlin, ((0, 0), (0, H - C))),
               batch.reshape(N, 1), blin.reshape(1, C), C)
    return out
